# Initial kernel scaffold; baseline (speedup 1.0000x reference)
import jax
import jax.numpy as jnp
from jax.experimental import pallas as pl


def kernel(x, edge_index, W_embed, W1, a1_src, a1_dst, W2, a2_src, a2_dst, W_dec):
    raise NotImplementedError("write your pallas kernel here")


# speedup vs baseline: 1.0852x; 1.0852x over previous
import jax
import jax.numpy as jnp
from jax.experimental import pallas as pl

HEADS = 4
HIDDEN = 64
EMB_OUT = 128


def _gat_conv(x, edge_index, W, a_src, a_dst, heads, out_ch, concat):
    n = x.shape[0]
    src = edge_index[0]
    dst = edge_index[1]
    h = (x @ W).reshape(n, heads, out_ch)
    alpha_src = (h * a_src[None]).sum(-1)
    alpha_dst = (h * a_dst[None]).sum(-1)
    e = alpha_src[src] + alpha_dst[dst]
    e = jax.nn.leaky_relu(e, 0.2)
    ex = jnp.exp(e)
    denom = jax.ops.segment_sum(ex, dst, num_segments=n)
    coef = ex / (denom[dst] + 1e-16)
    msg = h[src] * coef[:, :, None]
    out = jax.ops.segment_sum(msg, dst, num_segments=n)
    if concat:
        return out.reshape(n, heads * out_ch)
    return out.mean(axis=1)


def _matmul_kernel(x_ref, w_ref, o_ref):
    o_ref[...] = jnp.dot(x_ref[...], w_ref[...],
                         preferred_element_type=jnp.float32)


def _pallas_matmul(x, w):
    m, k = x.shape
    k2, n = w.shape
    return pl.pallas_call(
        _matmul_kernel,
        out_shape=jax.ShapeDtypeStruct((m, n), jnp.float32),
        grid=(m // 2000,),
        in_specs=[
            pl.BlockSpec((2000, k), lambda i: (i, 0)),
            pl.BlockSpec((k2, n), lambda i: (0, 0)),
        ],
        out_specs=pl.BlockSpec((2000, n), lambda i: (i, 0)),
    )(x, w)


def kernel(x, edge_index, W_embed, W1, a1_src, a1_dst, W2, a2_src, a2_dst, W_dec):
    h = jax.nn.relu(x @ W_embed)
    h = jax.nn.relu(_gat_conv(h, edge_index, W1, a1_src, a1_dst, HEADS, HIDDEN, True))
    h = jax.nn.relu(_gat_conv(h, edge_index, W2, a2_src, a2_dst, 1, EMB_OUT, False))
    return _pallas_matmul(h, W_dec)


# speedup vs baseline: 20.5196x; 18.9091x over previous
import dataclasses
import functools

import jax
import jax.numpy as jnp
from jax import lax
from jax.experimental import pallas as pl
from jax.experimental.pallas import tpu as pltpu
from jax.experimental.pallas import tpu_sc as plsc

_NT = 16
_C = 64


def _embed_kernel(x_ref, we_ref, w1_ref, a8_ref, h1_ref, al_ref):
    h0 = jnp.maximum(
        jnp.dot(x_ref[...], we_ref[...], preferred_element_type=jnp.float32), 0.0)
    h1 = jnp.dot(h0, w1_ref[...], preferred_element_type=jnp.float32)
    h1_ref[...] = h1
    al_ref[...] = jnp.dot(h1, a8_ref[...], preferred_element_type=jnp.float32)


def _stage_a(x, W_embed, W1, A8):
    n, d_in = x.shape
    d_e = W_embed.shape[1]
    d_h = W1.shape[1]
    blk = 1000
    return pl.pallas_call(
        _embed_kernel,
        grid=(n // blk,),
        in_specs=[
            pl.BlockSpec((blk, d_in), lambda i: (i, 0)),
            pl.BlockSpec((d_in, d_e), lambda i: (0, 0)),
            pl.BlockSpec((d_e, d_h), lambda i: (0, 0)),
            pl.BlockSpec((d_h, 8), lambda i: (0, 0)),
        ],
        out_specs=[
            pl.BlockSpec((blk, d_h), lambda i: (i, 0)),
            pl.BlockSpec((blk, 8), lambda i: (i, 0)),
        ],
        out_shape=[
            jax.ShapeDtypeStruct((n, d_h), jnp.float32),
            jax.ShapeDtypeStruct((n, 8), jnp.float32),
        ],
    )(x, W_embed, W1, A8)


def _mid_kernel(a_ref, d_ref, w_ref, o_ref):
    halves = []
    for ci in range(2):
        a = a_ref[ci]
        d = d_ref[ci]
        for j in range(2):
            den = d[:, j:j + 1]
            num = a[:, j * 64:(j + 1) * 64]
            halves.append(jnp.where(den > 0, num / den, 0.0))
    g = jnp.maximum(jnp.concatenate(halves, axis=1), 0.0)
    o_ref[...] = jnp.dot(g, w_ref[...], preferred_element_type=jnp.float32)


def _stage_b(acc1, den1, W2cat):
    npad = acc1.shape[1]
    blk = 1024
    return pl.pallas_call(
        _mid_kernel,
        grid=(npad // blk,),
        in_specs=[
            pl.BlockSpec((2, blk, 128), lambda i: (0, i, 0)),
            pl.BlockSpec((2, blk, 4), lambda i: (0, i, 0)),
            pl.BlockSpec((256, 130), lambda i: (0, 0)),
        ],
        out_specs=pl.BlockSpec((blk, 130), lambda i: (i, 0)),
        out_shape=jax.ShapeDtypeStruct((npad, 130), jnp.float32),
    )(acc1, den1, W2cat)


def _dec_kernel(a_ref, d_ref, wd_ref, o_ref):
    a = a_ref[0] + a_ref[1]
    den = d_ref[0][:, 0:1] + d_ref[1][:, 0:1]
    g = jnp.maximum(jnp.where(den > 0, a / den, 0.0), 0.0)
    o_ref[...] = jnp.dot(g, wd_ref[...], preferred_element_type=jnp.float32)


def _stage_c(acc2, den2, W_dec):
    npad = acc2.shape[1]
    n_cls = W_dec.shape[1]
    blk = 1024
    return pl.pallas_call(
        _dec_kernel,
        grid=(npad // blk,),
        in_specs=[
            pl.BlockSpec((2, blk, 128), lambda i: (0, i, 0)),
            pl.BlockSpec((2, blk, 4), lambda i: (0, i, 0)),
            pl.BlockSpec((128, n_cls), lambda i: (0, 0)),
        ],
        out_specs=pl.BlockSpec((blk, n_cls), lambda i: (i, 0)),
        out_shape=jax.ShapeDtypeStruct((npad, n_cls), jnp.float32),
    )(acc2, den2, W_dec)


def _make_sc_gat(n_pad, e_span, per_core, n_real):
    F = 128
    HPC = 2
    n32 = n_pad // 32
    per_tile = per_core // _NT
    n_chunks = per_tile // _C
    rows_per_tile = n_pad // _NT
    den_rows_per_tile = n32 // _NT
    mesh_v = plsc.VectorSubcoreMesh(core_axis_name="c", subcore_axis_name="s")
    cp = pltpu.CompilerParams()
    if "needs_layout_passes" in pltpu.CompilerParams.__dataclass_fields__:
        cp = dataclasses.replace(cp, needs_layout_passes=False)

    @functools.partial(
        pl.kernel,
        out_type=[
            jax.ShapeDtypeStruct((2, n_pad, F), jnp.float32),
            jax.ShapeDtypeStruct((2, n32, 128), jnp.float32),
        ],
        mesh=mesh_v,
        compiler_params=cp,
        scratch_types=[
            pltpu.VMEM((1, _C), jnp.int32),
            pltpu.VMEM((1, _C), jnp.int32),
            pltpu.VMEM((1, _C), jnp.int32),
            pltpu.VMEM((1, _C * HPC), jnp.int32),
            pltpu.VMEM((1, _C * HPC), jnp.int32),
            pltpu.VMEM((1, _C), jnp.int32),
            pltpu.VMEM((HPC, _C), jnp.int32),
            pltpu.VMEM((1, _C * HPC), jnp.float32),
            pltpu.VMEM((1, _C * HPC), jnp.float32),
            pltpu.VMEM((_C, F), jnp.float32),
            pltpu.VMEM((_C, 128), jnp.float32),
            pltpu.VMEM_SHARED((n_pad, F), jnp.float32),
            pltpu.VMEM_SHARED((n32, 128), jnp.float32),
        ],
    )
    def sc_gat(hf_hbm, asrc_hbm, adst_hbm, src_hbm, dst_hbm,
               acc_out, den_out,
               sbuf, sobuf, dbuf, sibuf, dibuf, d8buf, colbuf,
               abuf_s, abuf_d, gbuf, expad, accS, denS):
        c = lax.axis_index("c")
        s = lax.axis_index("s")
        coff = c * n_pad

        lanes = lax.iota(jnp.int32, 16)
        zeros16 = jnp.zeros((16,), jnp.float32)
        izeros16 = jnp.zeros((16,), jnp.int32)

        for r in range(_C):
            for q in range(8):
                expad[r, pl.ds(q * 16, 16)] = zeros16
            for q in range(F // 16):
                gbuf[r, pl.ds(q * 16, 16)] = zeros16
        for g in range(_C // 16):
            for j in range(HPC):
                colbuf[j, pl.ds(g * 16, 16)] = izeros16
        for b in range(rows_per_tile // _C):
            r0 = s * rows_per_tile + b * _C
            pltpu.sync_copy(gbuf, accS.at[pl.ds(r0, _C)])

        den_chunks = []
        d0 = 0
        while d0 < n32:
            dlen = min(_C, n32 - d0)
            den_chunks.append((d0, dlen))
            d0 += dlen

        @pl.when(s == 0)
        def _zero_den():
            for r0, rl in den_chunks:
                pltpu.sync_copy(expad.at[pl.ds(0, rl)], denS.at[pl.ds(r0, rl)])

        plsc.subcore_barrier()

        @pl.loop(0, n_chunks)
        def _chunk(i):
            base = c * e_span + s * per_tile + i * _C
            pltpu.sync_copy(src_hbm.at[pl.ds(base, _C)], sbuf.at[0])
            pltpu.sync_copy(dst_hbm.at[pl.ds(base, _C)], dbuf.at[0])
            for g in range(_C // 16):
                sv = sbuf[0, pl.ds(g * 16, 16)]
                dv = dbuf[0, pl.ds(g * 16, 16)]
                rows = g * 16 + lanes
                so = sv + coff
                do = dv + coff
                sobuf[0, pl.ds(g * 16, 16)] = so
                d8buf[0, pl.ds(g * 16, 16)] = lax.shift_right_logical(dv, 5)
                if HPC > 1:
                    for j in range(HPC):
                        plsc.store_scatter(
                            sibuf, [jnp.zeros((16,), jnp.int32),
                                    rows * HPC + j], so * HPC + j)
                        plsc.store_scatter(
                            dibuf, [jnp.zeros((16,), jnp.int32),
                                    rows * HPC + j], do * HPC + j)
                else:
                    sibuf[0, pl.ds(g * 16, 16)] = so
                    dibuf[0, pl.ds(g * 16, 16)] = do
            pltpu.sync_copy(hf_hbm.at[sobuf.at[0]], gbuf)
            pltpu.sync_copy(asrc_hbm.at[sibuf.at[0]], abuf_s.at[0])
            pltpu.sync_copy(adst_hbm.at[dibuf.at[0]], abuf_d.at[0])
            for g in range(_C // 16):
                dv = dbuf[0, pl.ds(g * 16, 16)]
                rows = g * 16 + lanes
                for j in range(HPC):
                    jv = jnp.full((16,), j, jnp.int32)
                    zr = jnp.zeros((16,), jnp.int32)
                    a_s = plsc.load_gather(abuf_s, [zr, rows * HPC + j])
                    a_d = plsc.load_gather(abuf_d, [zr, rows * HPC + j])
                    e = a_s + a_d
                    e = jnp.where(e > 0, e, e * 0.2)
                    ex = jnp.exp(e)
                    oldcol = colbuf[j, pl.ds(g * 16, 16)]
                    plsc.store_scatter(expad, [rows, oldcol], zeros16)
                    newcol = (dv & 31) * 4 + j
                    colbuf[j, pl.ds(g * 16, 16)] = newcol
                    plsc.store_scatter(expad, [rows, newcol], ex)
                    for i16 in range(16):
                        row = g * 16 + i16
                        sc = ex[i16]
                        for q in range(4):
                            col = j * 64 + q * 16
                            gbuf[row, pl.ds(col, 16)] = (
                                gbuf[row, pl.ds(col, 16)] * sc)
            pltpu.sync_copy(gbuf, accS.at[dbuf.at[0]], add=True)
            pltpu.sync_copy(expad, denS.at[d8buf.at[0]], add=True)

        plsc.subcore_barrier()

        for b in range(rows_per_tile // _C):
            r0 = s * rows_per_tile + b * _C
            pltpu.sync_copy(accS.at[pl.ds(r0, _C)], gbuf)
            pltpu.sync_copy(gbuf, acc_out.at[c, pl.ds(r0, _C)])

        @pl.when(s == 0)
        def _flush_den():
            for r0, rl in den_chunks:
                pltpu.sync_copy(denS.at[pl.ds(r0, rl)], expad.at[pl.ds(0, rl)])
                pltpu.sync_copy(expad.at[pl.ds(0, rl)],
                                den_out.at[c, pl.ds(r0, rl)])

    return sc_gat


def kernel(x, edge_index, W_embed, W1, a1_src, a1_dst, W2, a2_src, a2_dst, W_dec):
    n, _ = x.shape
    e = edge_index.shape[1]
    heads, hid = a1_src.shape
    emb_out = W2.shape[1]

    n_pad = (n // 2048 + 1) * 2048
    e_pad = ((e + 2047) // 2048) * 2048

    A_src = jax.scipy.linalg.block_diag(*[a1_src[h][:, None] for h in range(heads)])
    A_dst = jax.scipy.linalg.block_diag(*[a1_dst[h][:, None] for h in range(heads)])
    A8 = jnp.concatenate([A_src, A_dst], axis=1)
    W2cat = jnp.concatenate([W2, W2 @ a2_src.T, W2 @ a2_dst.T], axis=1)

    pad_e = e_pad - e
    pad_idx = n + jnp.arange(pad_e, dtype=jnp.int32) % (n_pad - n)
    src_p = jnp.concatenate([edge_index[0], pad_idx])
    dst_p = jnp.concatenate([edge_index[1], pad_idx])

    h1, al8 = _stage_a(x, W_embed, W1, A8)
    pad_n = ((0, 0), (0, n_pad - n), (0, 0))
    hf1 = jnp.pad(h1.reshape(n, 2, 128).transpose(1, 0, 2), pad_n)
    hf1 = hf1.reshape(2 * n_pad, 128)
    asrc1 = jnp.pad(al8[:, :4].reshape(n, 2, 2).transpose(1, 0, 2), pad_n)
    asrc1 = asrc1.reshape(2 * n_pad * 2)
    adst1 = jnp.pad(al8[:, 4:8].reshape(n, 2, 2).transpose(1, 0, 2), pad_n)
    adst1 = adst1.reshape(2 * n_pad * 2)

    src2x = jnp.concatenate([src_p, src_p])
    dst2x = jnp.concatenate([dst_p, dst_p])
    sc1 = _make_sc_gat(n_pad, e_pad, e_pad, n)
    acc1, den1 = sc1(hf1, asrc1, adst1, src2x, dst2x)
    den1v = den1.reshape(2, n_pad, 4)

    h2cat = _stage_b(acc1, den1v, W2cat)
    h2 = h2cat[:, :emb_out]
    hf2 = jnp.concatenate([h2, h2], axis=0)
    a2s = jnp.repeat(h2cat[:, emb_out], 2)
    a2d = jnp.repeat(h2cat[:, emb_out + 1], 2)
    asrc2 = jnp.concatenate([a2s, a2s])
    adst2 = jnp.concatenate([a2d, a2d])

    sc2 = _make_sc_gat(n_pad, e_pad // 2, e_pad // 2, n)
    acc2, den2 = sc2(hf2, asrc2, adst2, src_p, dst_p)
    den2v = den2.reshape(2, n_pad, 4)

    out = _stage_c(acc2, den2v, W_dec)
    return out[:n]


# speedup vs baseline: 38.0497x; 1.8543x over previous
import dataclasses
import functools

import jax
import jax.numpy as jnp
from jax import lax
from jax.experimental import pallas as pl
from jax.experimental.pallas import tpu as pltpu
from jax.experimental.pallas import tpu_sc as plsc

_NT = 16
_C = 64


def _embed_kernel(x_ref, we_ref, w1_ref, a8_ref, h1_ref, al_ref):
    h0 = jnp.maximum(
        jnp.dot(x_ref[...], we_ref[...], preferred_element_type=jnp.float32), 0.0)
    h1 = jnp.dot(h0, w1_ref[...], preferred_element_type=jnp.float32)
    h1_ref[...] = h1
    al_ref[...] = jnp.dot(h1, a8_ref[...], preferred_element_type=jnp.float32)


def _stage_a(x, W_embed, W1, A8):
    n, d_in = x.shape
    d_e = W_embed.shape[1]
    d_h = W1.shape[1]
    blk = 1000
    return pl.pallas_call(
        _embed_kernel,
        grid=(n // blk,),
        in_specs=[
            pl.BlockSpec((blk, d_in), lambda i: (i, 0)),
            pl.BlockSpec((d_in, d_e), lambda i: (0, 0)),
            pl.BlockSpec((d_e, d_h), lambda i: (0, 0)),
            pl.BlockSpec((d_h, 8), lambda i: (0, 0)),
        ],
        out_specs=[
            pl.BlockSpec((blk, d_h), lambda i: (i, 0)),
            pl.BlockSpec((blk, 8), lambda i: (i, 0)),
        ],
        out_shape=[
            jax.ShapeDtypeStruct((n, d_h), jnp.float32),
            jax.ShapeDtypeStruct((n, 8), jnp.float32),
        ],
    )(x, W_embed, W1, A8)


def _mid_kernel(a_ref, d_ref, w_ref, o_ref):
    halves = []
    for ci in range(2):
        a = a_ref[ci]
        d = d_ref[ci]
        for j in range(2):
            den = d[:, j:j + 1]
            num = a[:, j * 64:(j + 1) * 64]
            halves.append(jnp.where(den > 0, num / den, 0.0))
    g = jnp.maximum(jnp.concatenate(halves, axis=1), 0.0)
    o_ref[...] = jnp.dot(g, w_ref[...], preferred_element_type=jnp.float32)


def _stage_b(acc1, den1, W2cat):
    npad = acc1.shape[1]
    blk = 1024
    return pl.pallas_call(
        _mid_kernel,
        grid=(npad // blk,),
        in_specs=[
            pl.BlockSpec((2, blk, 128), lambda i: (0, i, 0)),
            pl.BlockSpec((2, blk, 4), lambda i: (0, i, 0)),
            pl.BlockSpec((256, 130), lambda i: (0, 0)),
        ],
        out_specs=pl.BlockSpec((blk, 130), lambda i: (i, 0)),
        out_shape=jax.ShapeDtypeStruct((npad, 130), jnp.float32),
    )(acc1, den1, W2cat)


def _dec_kernel(a_ref, d_ref, wd_ref, o_ref):
    a = a_ref[0] + a_ref[1]
    den = d_ref[0][:, 0:1] + d_ref[1][:, 0:1]
    g = jnp.maximum(jnp.where(den > 0, a / den, 0.0), 0.0)
    o_ref[...] = jnp.dot(g, wd_ref[...], preferred_element_type=jnp.float32)


def _stage_c(acc2, den2, W_dec):
    npad = acc2.shape[1]
    n_cls = W_dec.shape[1]
    blk = 1024
    return pl.pallas_call(
        _dec_kernel,
        grid=(npad // blk,),
        in_specs=[
            pl.BlockSpec((2, blk, 128), lambda i: (0, i, 0)),
            pl.BlockSpec((2, blk, 4), lambda i: (0, i, 0)),
            pl.BlockSpec((128, n_cls), lambda i: (0, 0)),
        ],
        out_specs=pl.BlockSpec((blk, n_cls), lambda i: (i, 0)),
        out_shape=jax.ShapeDtypeStruct((npad, n_cls), jnp.float32),
    )(acc2, den2, W_dec)


def _make_sc_gat(n_pad, e_span, per_core, n_real):
    F = 128
    HPC = 2
    n32 = n_pad // 32
    per_tile = per_core // _NT
    n_chunks = per_tile // _C
    rows_per_tile = n_pad // _NT
    den_rows_per_tile = n32 // _NT
    mesh_v = plsc.VectorSubcoreMesh(core_axis_name="c", subcore_axis_name="s")
    cp = pltpu.CompilerParams()
    if "needs_layout_passes" in pltpu.CompilerParams.__dataclass_fields__:
        cp = dataclasses.replace(cp, needs_layout_passes=False)

    @functools.partial(
        pl.kernel,
        out_type=[
            jax.ShapeDtypeStruct((2, n_pad, F), jnp.float32),
            jax.ShapeDtypeStruct((2, n32, 128), jnp.float32),
        ],
        mesh=mesh_v,
        compiler_params=cp,
        scratch_types=(
            [pltpu.VMEM((1, _C), jnp.int32)] * 2
            + [pltpu.VMEM((1, _C), jnp.int32)] * 2
            + [pltpu.VMEM((1, _C), jnp.int32)] * 2
            + [pltpu.VMEM((1, _C * HPC), jnp.int32)] * 2
            + [pltpu.VMEM((1, _C * HPC), jnp.int32)] * 2
            + [pltpu.VMEM((1, _C), jnp.int32)] * 2
            + [pltpu.VMEM((1, _C * HPC), jnp.float32)] * 2
            + [pltpu.VMEM((1, _C * HPC), jnp.float32)] * 2
            + [pltpu.VMEM((_C, F), jnp.float32)] * 2
            + [pltpu.SemaphoreType.DMA] * 2
            + [
                pltpu.VMEM((HPC, _C), jnp.int32),
                pltpu.VMEM((_C, 128), jnp.float32),
                pltpu.VMEM_SHARED((n_pad, F), jnp.float32),
                pltpu.VMEM_SHARED((n32, 128), jnp.float32),
            ]
        ),
    )
    def sc_gat(hf_hbm, asrc_hbm, adst_hbm, src_hbm, dst_hbm,
               acc_out, den_out,
               sbuf0, sbuf1, sobuf0, sobuf1, dbuf0, dbuf1,
               sibuf0, sibuf1, dibuf0, dibuf1, d8buf0, d8buf1,
               abuf_s0, abuf_s1, abuf_d0, abuf_d1, gbuf0, gbuf1,
               sem0, sem1, colbuf, expad, accS, denS):
        bufsets = (
            (sbuf0, sobuf0, dbuf0, sibuf0, dibuf0, d8buf0,
             abuf_s0, abuf_d0, gbuf0, sem0),
            (sbuf1, sobuf1, dbuf1, sibuf1, dibuf1, d8buf1,
             abuf_s1, abuf_d1, gbuf1, sem1),
        )
        gbuf = gbuf0
        c = lax.axis_index("c")
        s = lax.axis_index("s")
        coff = c * n_pad

        lanes = lax.iota(jnp.int32, 16)
        zeros16 = jnp.zeros((16,), jnp.float32)
        izeros16 = jnp.zeros((16,), jnp.int32)

        for r in range(_C):
            for q in range(8):
                expad[r, pl.ds(q * 16, 16)] = zeros16
            for q in range(F // 16):
                gbuf[r, pl.ds(q * 16, 16)] = zeros16
        for g in range(_C // 16):
            for j in range(HPC):
                colbuf[j, pl.ds(g * 16, 16)] = izeros16
        for b in range(rows_per_tile // _C):
            r0 = s * rows_per_tile + b * _C
            pltpu.sync_copy(gbuf, accS.at[pl.ds(r0, _C)])

        den_chunks = []
        d0 = 0
        while d0 < n32:
            dlen = min(_C, n32 - d0)
            den_chunks.append((d0, dlen))
            d0 += dlen

        @pl.when(s == 0)
        def _zero_den():
            for r0, rl in den_chunks:
                pltpu.sync_copy(expad.at[pl.ds(0, rl)], denS.at[pl.ds(r0, rl)])

        plsc.subcore_barrier()

        def _gather_descs(B):
            (sb, sob, db, sib, dib, d8b, abs_, abd, gb, sem) = B
            return (pltpu.make_async_copy(hf_hbm.at[sob.at[0]], gb, sem),
                    pltpu.make_async_copy(asrc_hbm.at[sib.at[0]],
                                          abs_.at[0], sem),
                    pltpu.make_async_copy(adst_hbm.at[dib.at[0]],
                                          abd.at[0], sem))

        def _prefetch(i, B):
            (sb, sob, db, sib, dib, d8b, abs_, abd, gb, sem) = B
            base = c * e_span + s * per_tile + i * _C
            pltpu.sync_copy(src_hbm.at[pl.ds(base, _C)], sb.at[0])
            pltpu.sync_copy(dst_hbm.at[pl.ds(base, _C)], db.at[0])
            for g in range(_C // 16):
                sv = sb[0, pl.ds(g * 16, 16)]
                dv = db[0, pl.ds(g * 16, 16)]
                rows = g * 16 + lanes
                so = sv + coff
                do = dv + coff
                sob[0, pl.ds(g * 16, 16)] = so
                d8b[0, pl.ds(g * 16, 16)] = lax.shift_right_logical(dv, 5)
                for j in range(HPC):
                    plsc.store_scatter(
                        sib, [jnp.zeros((16,), jnp.int32),
                              rows * HPC + j], so * HPC + j)
                    plsc.store_scatter(
                        dib, [jnp.zeros((16,), jnp.int32),
                              rows * HPC + j], do * HPC + j)
            for d in _gather_descs(B):
                d.start()

        def _process(B):
            (sb, sob, db, sib, dib, d8b, abs_, abd, gb, sem) = B
            for d in _gather_descs(B):
                d.wait()
            for g in range(_C // 16):
                dv = db[0, pl.ds(g * 16, 16)]
                rows = g * 16 + lanes
                for j in range(HPC):
                    zr = jnp.zeros((16,), jnp.int32)
                    a_s = plsc.load_gather(abs_, [zr, rows * HPC + j])
                    a_d = plsc.load_gather(abd, [zr, rows * HPC + j])
                    e = a_s + a_d
                    e = jnp.where(e > 0, e, e * 0.2)
                    ex = jnp.exp(e)
                    oldcol = colbuf[j, pl.ds(g * 16, 16)]
                    plsc.store_scatter(expad, [rows, oldcol], zeros16)
                    newcol = (dv & 31) * 4 + j
                    colbuf[j, pl.ds(g * 16, 16)] = newcol
                    plsc.store_scatter(expad, [rows, newcol], ex)
                    for i16 in range(16):
                        row = g * 16 + i16
                        sc = ex[i16]
                        for q in range(4):
                            col = j * 64 + q * 16
                            gb[row, pl.ds(col, 16)] = (
                                gb[row, pl.ds(col, 16)] * sc)
            pltpu.sync_copy(gb, accS.at[db.at[0]], add=True)
            pltpu.sync_copy(expad, denS.at[d8b.at[0]], add=True)

        _prefetch(0, bufsets[0])

        @pl.loop(0, n_chunks, step=2)
        def _chunk(i):
            _prefetch(i + 1, bufsets[1])
            _process(bufsets[0])

            @pl.when(i + 2 < n_chunks)
            def _pf():
                _prefetch(i + 2, bufsets[0])

            _process(bufsets[1])

        plsc.subcore_barrier()

        for b in range(rows_per_tile // _C):
            r0 = s * rows_per_tile + b * _C
            pltpu.sync_copy(accS.at[pl.ds(r0, _C)], gbuf)
            pltpu.sync_copy(gbuf, acc_out.at[c, pl.ds(r0, _C)])

        @pl.when(s == 0)
        def _flush_den():
            for r0, rl in den_chunks:
                pltpu.sync_copy(denS.at[pl.ds(r0, rl)], expad.at[pl.ds(0, rl)])
                pltpu.sync_copy(expad.at[pl.ds(0, rl)],
                                den_out.at[c, pl.ds(r0, rl)])

    return sc_gat


def kernel(x, edge_index, W_embed, W1, a1_src, a1_dst, W2, a2_src, a2_dst, W_dec):
    n, _ = x.shape
    e = edge_index.shape[1]
    heads, hid = a1_src.shape
    emb_out = W2.shape[1]

    n_pad = (n // 2048 + 1) * 2048
    e_pad = ((e + 4095) // 4096) * 4096

    A_src = jax.scipy.linalg.block_diag(*[a1_src[h][:, None] for h in range(heads)])
    A_dst = jax.scipy.linalg.block_diag(*[a1_dst[h][:, None] for h in range(heads)])
    A8 = jnp.concatenate([A_src, A_dst], axis=1)
    W2cat = jnp.concatenate([W2, W2 @ a2_src.T, W2 @ a2_dst.T], axis=1)

    pad_e = e_pad - e
    pad_idx = n + jnp.arange(pad_e, dtype=jnp.int32) % (n_pad - n)
    src_p = jnp.concatenate([edge_index[0], pad_idx])
    dst_p = jnp.concatenate([edge_index[1], pad_idx])

    h1, al8 = _stage_a(x, W_embed, W1, A8)
    pad_n = ((0, 0), (0, n_pad - n), (0, 0))
    hf1 = jnp.pad(h1.reshape(n, 2, 128).transpose(1, 0, 2), pad_n)
    hf1 = hf1.reshape(2 * n_pad, 128)
    asrc1 = jnp.pad(al8[:, :4].reshape(n, 2, 2).transpose(1, 0, 2), pad_n)
    asrc1 = asrc1.reshape(2 * n_pad * 2)
    adst1 = jnp.pad(al8[:, 4:8].reshape(n, 2, 2).transpose(1, 0, 2), pad_n)
    adst1 = adst1.reshape(2 * n_pad * 2)

    src2x = jnp.concatenate([src_p, src_p])
    dst2x = jnp.concatenate([dst_p, dst_p])
    sc1 = _make_sc_gat(n_pad, e_pad, e_pad, n)
    acc1, den1 = sc1(hf1, asrc1, adst1, src2x, dst2x)
    den1v = den1.reshape(2, n_pad, 4)

    h2cat = _stage_b(acc1, den1v, W2cat)
    h2 = h2cat[:, :emb_out]
    hf2 = jnp.concatenate([h2, h2], axis=0)
    a2s = jnp.repeat(h2cat[:, emb_out], 2)
    a2d = jnp.repeat(h2cat[:, emb_out + 1], 2)
    asrc2 = jnp.concatenate([a2s, a2s])
    adst2 = jnp.concatenate([a2d, a2d])

    sc2 = _make_sc_gat(n_pad, e_pad // 2, e_pad // 2, n)
    acc2, den2 = sc2(hf2, asrc2, adst2, src_p, dst_p)
    den2v = den2.reshape(2, n_pad, 4)

    out = _stage_c(acc2, den2v, W_dec)
    return out[:n]


# speedup vs baseline: 42.1441x; 1.1076x over previous
import dataclasses
import functools

import jax
import jax.numpy as jnp
from jax import lax
from jax.experimental import pallas as pl
from jax.experimental.pallas import tpu as pltpu
from jax.experimental.pallas import tpu_sc as plsc

_NT = 16
_C = 64


def _embed_kernel(x_ref, we_ref, w1_ref, a8_ref, h1_ref, al_ref):
    h0 = jnp.maximum(
        jnp.dot(x_ref[...], we_ref[...], preferred_element_type=jnp.float32), 0.0)
    h1 = jnp.dot(h0, w1_ref[...], preferred_element_type=jnp.float32)
    h1_ref[...] = h1
    al_ref[...] = jnp.dot(h1, a8_ref[...], preferred_element_type=jnp.float32)


def _stage_a(x, W_embed, W1, A8):
    n, d_in = x.shape
    d_e = W_embed.shape[1]
    d_h = W1.shape[1]
    blk = 1000
    return pl.pallas_call(
        _embed_kernel,
        grid=(n // blk,),
        in_specs=[
            pl.BlockSpec((blk, d_in), lambda i: (i, 0)),
            pl.BlockSpec((d_in, d_e), lambda i: (0, 0)),
            pl.BlockSpec((d_e, d_h), lambda i: (0, 0)),
            pl.BlockSpec((d_h, 8), lambda i: (0, 0)),
        ],
        out_specs=[
            pl.BlockSpec((blk, d_h), lambda i: (i, 0)),
            pl.BlockSpec((blk, 8), lambda i: (i, 0)),
        ],
        out_shape=[
            jax.ShapeDtypeStruct((n, d_h), jnp.float32),
            jax.ShapeDtypeStruct((n, 8), jnp.float32),
        ],
    )(x, W_embed, W1, A8)


def _mid_kernel(a_ref, d_ref, w_ref, o_ref):
    halves = []
    for ci in range(2):
        a = a_ref[ci]
        d = d_ref[ci]
        for j in range(2):
            den = d[:, j:j + 1]
            num = a[:, j * 64:(j + 1) * 64]
            halves.append(jnp.where(den > 0, num / den, 0.0))
    g = jnp.maximum(jnp.concatenate(halves, axis=1), 0.0)
    o_ref[...] = jnp.dot(g, w_ref[...], preferred_element_type=jnp.float32)


def _stage_b(acc1, den1, W2cat):
    npad = acc1.shape[1]
    blk = 1024
    return pl.pallas_call(
        _mid_kernel,
        grid=(npad // blk,),
        in_specs=[
            pl.BlockSpec((2, blk, 128), lambda i: (0, i, 0)),
            pl.BlockSpec((2, blk, 4), lambda i: (0, i, 0)),
            pl.BlockSpec((256, 130), lambda i: (0, 0)),
        ],
        out_specs=pl.BlockSpec((blk, 130), lambda i: (i, 0)),
        out_shape=jax.ShapeDtypeStruct((npad, 130), jnp.float32),
    )(acc1, den1, W2cat)


def _dec_kernel(a_ref, d_ref, wd_ref, o_ref):
    a = a_ref[0] + a_ref[1]
    den = d_ref[0][:, 0:1] + d_ref[1][:, 0:1]
    g = jnp.maximum(jnp.where(den > 0, a / den, 0.0), 0.0)
    o_ref[...] = jnp.dot(g, wd_ref[...], preferred_element_type=jnp.float32)


def _stage_c(acc2, den2, W_dec):
    npad = acc2.shape[1]
    n_cls = W_dec.shape[1]
    blk = 1024
    return pl.pallas_call(
        _dec_kernel,
        grid=(npad // blk,),
        in_specs=[
            pl.BlockSpec((2, blk, 128), lambda i: (0, i, 0)),
            pl.BlockSpec((2, blk, 4), lambda i: (0, i, 0)),
            pl.BlockSpec((128, n_cls), lambda i: (0, 0)),
        ],
        out_specs=pl.BlockSpec((blk, n_cls), lambda i: (i, 0)),
        out_shape=jax.ShapeDtypeStruct((npad, n_cls), jnp.float32),
    )(acc2, den2, W_dec)


def _make_sc_gat(n_pad, e_span, per_core, n_real):
    F = 128
    HPC = 2
    n32 = n_pad // 32
    per_tile = per_core // _NT
    n_chunks = per_tile // _C
    rows_per_tile = n_pad // _NT
    den_rows_per_tile = n32 // _NT
    mesh_v = plsc.VectorSubcoreMesh(core_axis_name="c", subcore_axis_name="s")
    cp = pltpu.CompilerParams()
    if "needs_layout_passes" in pltpu.CompilerParams.__dataclass_fields__:
        cp = dataclasses.replace(cp, needs_layout_passes=False)

    @functools.partial(
        pl.kernel,
        out_type=[
            jax.ShapeDtypeStruct((2, n_pad, F), jnp.float32),
            jax.ShapeDtypeStruct((2, n32, 128), jnp.float32),
        ],
        mesh=mesh_v,
        compiler_params=cp,
        scratch_types=(
            [pltpu.VMEM((2, _C), jnp.int32)] * 2
            + [pltpu.VMEM((1, _C), jnp.int32)] * 2
            + [pltpu.VMEM((1, _C * HPC), jnp.int32)] * 2
            + [pltpu.VMEM((1, _C * HPC), jnp.int32)] * 2
            + [pltpu.VMEM((1, _C), jnp.int32)] * 2
            + [pltpu.VMEM((1, _C * HPC), jnp.float32)] * 2
            + [pltpu.VMEM((1, _C * HPC), jnp.float32)] * 2
            + [pltpu.VMEM((_C, F), jnp.float32)] * 2
            + [pltpu.SemaphoreType.DMA] * 2
            + [
                pltpu.VMEM((HPC, _C), jnp.int32),
                pltpu.VMEM((_C, 128), jnp.float32),
                pltpu.VMEM_SHARED((n_pad, F), jnp.float32),
                pltpu.VMEM_SHARED((n32, 128), jnp.float32),
            ]
        ),
    )
    def sc_gat(hf_hbm, asrc_hbm, adst_hbm, e2_hbm,
               acc_out, den_out,
               sdbuf0, sdbuf1, sobuf0, sobuf1,
               sibuf0, sibuf1, dibuf0, dibuf1, d8buf0, d8buf1,
               abuf_s0, abuf_s1, abuf_d0, abuf_d1, gbuf0, gbuf1,
               sem0, sem1, colbuf, expad, accS, denS):
        bufsets = (
            (sdbuf0, sobuf0, sibuf0, dibuf0, d8buf0,
             abuf_s0, abuf_d0, gbuf0, sem0),
            (sdbuf1, sobuf1, sibuf1, dibuf1, d8buf1,
             abuf_s1, abuf_d1, gbuf1, sem1),
        )
        gbuf = gbuf0
        c = lax.axis_index("c")
        s = lax.axis_index("s")
        coff = c * n_pad

        lanes = lax.iota(jnp.int32, 16)
        zeros16 = jnp.zeros((16,), jnp.float32)
        izeros16 = jnp.zeros((16,), jnp.int32)

        for r in range(_C):
            for q in range(8):
                expad[r, pl.ds(q * 16, 16)] = zeros16
            for q in range(F // 16):
                gbuf[r, pl.ds(q * 16, 16)] = zeros16
        for g in range(_C // 16):
            for j in range(HPC):
                colbuf[j, pl.ds(g * 16, 16)] = izeros16
        for b in range(rows_per_tile // _C):
            r0 = s * rows_per_tile + b * _C
            pltpu.sync_copy(gbuf, accS.at[pl.ds(r0, _C)])

        den_chunks = []
        d0 = 0
        while d0 < n32:
            dlen = min(_C, n32 - d0)
            den_chunks.append((d0, dlen))
            d0 += dlen

        @pl.when(s == 0)
        def _zero_den():
            for r0, rl in den_chunks:
                pltpu.sync_copy(expad.at[pl.ds(0, rl)], denS.at[pl.ds(r0, rl)])

        plsc.subcore_barrier()

        def _gather_descs(B):
            (sdb, sob, sib, dib, d8b, abs_, abd, gb, sem) = B
            return (pltpu.make_async_copy(hf_hbm.at[sob.at[0]], gb, sem),
                    pltpu.make_async_copy(asrc_hbm.at[sib.at[0]],
                                          abs_.at[0], sem),
                    pltpu.make_async_copy(adst_hbm.at[dib.at[0]],
                                          abd.at[0], sem))

        def _prefetch(i, B):
            (sdb, sob, sib, dib, d8b, abs_, abd, gb, sem) = B
            chunkid = c * (e_span // _C) + s * (per_tile // _C) + i
            pltpu.sync_copy(e2_hbm.at[chunkid], sdb)
            for g in range(_C // 16):
                sv = sdb[0, pl.ds(g * 16, 16)]
                dv = sdb[1, pl.ds(g * 16, 16)]
                rows = g * 16 + lanes
                so = sv + coff
                do = dv + coff
                sob[0, pl.ds(g * 16, 16)] = so
                d8b[0, pl.ds(g * 16, 16)] = lax.shift_right_logical(dv, 5)
                for j in range(HPC):
                    plsc.store_scatter(
                        sib, [jnp.zeros((16,), jnp.int32),
                              rows * HPC + j], so * HPC + j)
                    plsc.store_scatter(
                        dib, [jnp.zeros((16,), jnp.int32),
                              rows * HPC + j], do * HPC + j)
            for d in _gather_descs(B):
                d.start()

        def _process(B):
            (sdb, sob, sib, dib, d8b, abs_, abd, gb, sem) = B
            for d in _gather_descs(B):
                d.wait()
            for g in range(_C // 16):
                dv = sdb[1, pl.ds(g * 16, 16)]
                rows = g * 16 + lanes
                for j in range(HPC):
                    zr = jnp.zeros((16,), jnp.int32)
                    a_s = plsc.load_gather(abs_, [zr, rows * HPC + j])
                    a_d = plsc.load_gather(abd, [zr, rows * HPC + j])
                    e = a_s + a_d
                    e = jnp.where(e > 0, e, e * 0.2)
                    ex = jnp.exp(e)
                    oldcol = colbuf[j, pl.ds(g * 16, 16)]
                    plsc.store_scatter(expad, [rows, oldcol], zeros16)
                    newcol = (dv & 31) * 4 + j
                    colbuf[j, pl.ds(g * 16, 16)] = newcol
                    plsc.store_scatter(expad, [rows, newcol], ex)
                    for i16 in range(16):
                        row = g * 16 + i16
                        sc = ex[i16]
                        for q in range(4):
                            col = j * 64 + q * 16
                            gb[row, pl.ds(col, 16)] = (
                                gb[row, pl.ds(col, 16)] * sc)
            pltpu.sync_copy(gb, accS.at[sdb.at[1]], add=True)
            pltpu.sync_copy(expad, denS.at[d8b.at[0]], add=True)

        _prefetch(0, bufsets[0])

        @pl.loop(0, n_chunks, step=2)
        def _chunk(i):
            _prefetch(i + 1, bufsets[1])
            _process(bufsets[0])

            @pl.when(i + 2 < n_chunks)
            def _pf():
                _prefetch(i + 2, bufsets[0])

            _process(bufsets[1])

        plsc.subcore_barrier()

        for b in range(rows_per_tile // _C):
            r0 = s * rows_per_tile + b * _C
            pltpu.sync_copy(accS.at[pl.ds(r0, _C)], gbuf)
            pltpu.sync_copy(gbuf, acc_out.at[c, pl.ds(r0, _C)])

        @pl.when(s == 0)
        def _flush_den():
            for r0, rl in den_chunks:
                pltpu.sync_copy(denS.at[pl.ds(r0, rl)], expad.at[pl.ds(0, rl)])
                pltpu.sync_copy(expad.at[pl.ds(0, rl)],
                                den_out.at[c, pl.ds(r0, rl)])

    return sc_gat


def kernel(x, edge_index, W_embed, W1, a1_src, a1_dst, W2, a2_src, a2_dst, W_dec):
    n, _ = x.shape
    e = edge_index.shape[1]
    heads, hid = a1_src.shape
    emb_out = W2.shape[1]

    n_pad = (n // 2048 + 1) * 2048
    e_pad = ((e + 4095) // 4096) * 4096

    A_src = jax.scipy.linalg.block_diag(*[a1_src[h][:, None] for h in range(heads)])
    A_dst = jax.scipy.linalg.block_diag(*[a1_dst[h][:, None] for h in range(heads)])
    A8 = jnp.concatenate([A_src, A_dst], axis=1)
    W2cat = jnp.concatenate([W2, W2 @ a2_src.T, W2 @ a2_dst.T], axis=1)

    pad_e = e_pad - e
    pad_idx = n + jnp.arange(pad_e, dtype=jnp.int32) % (n_pad - n)
    src_p = jnp.concatenate([edge_index[0], pad_idx])
    dst_p = jnp.concatenate([edge_index[1], pad_idx])

    h1, al8 = _stage_a(x, W_embed, W1, A8)
    pad_n = ((0, 0), (0, n_pad - n), (0, 0))
    hf1 = jnp.pad(h1.reshape(n, 2, 128).transpose(1, 0, 2), pad_n)
    hf1 = hf1.reshape(2 * n_pad, 128)
    asrc1 = jnp.pad(al8[:, :4].reshape(n, 2, 2).transpose(1, 0, 2), pad_n)
    asrc1 = asrc1.reshape(2 * n_pad * 2)
    adst1 = jnp.pad(al8[:, 4:8].reshape(n, 2, 2).transpose(1, 0, 2), pad_n)
    adst1 = adst1.reshape(2 * n_pad * 2)

    e2 = jnp.stack([src_p.reshape(-1, 64), dst_p.reshape(-1, 64)], axis=1)
    e2x = jnp.concatenate([e2, e2])
    sc1 = _make_sc_gat(n_pad, e_pad, e_pad, n)
    acc1, den1 = sc1(hf1, asrc1, adst1, e2x)
    den1v = den1.reshape(2, n_pad, 4)

    h2cat = _stage_b(acc1, den1v, W2cat)
    h2 = h2cat[:, :emb_out]
    hf2 = jnp.concatenate([h2, h2], axis=0)
    a2s = jnp.repeat(h2cat[:, emb_out], 2)
    a2d = jnp.repeat(h2cat[:, emb_out + 1], 2)
    asrc2 = jnp.concatenate([a2s, a2s])
    adst2 = jnp.concatenate([a2d, a2d])

    sc2 = _make_sc_gat(n_pad, e_pad // 2, e_pad // 2, n)
    acc2, den2 = sc2(hf2, asrc2, adst2, e2)
    den2v = den2.reshape(2, n_pad, 4)

    out = _stage_c(acc2, den2v, W_dec)
    return out[:n]
